# Initial kernel scaffold; baseline (speedup 1.0000x reference)
import jax
import jax.numpy as jnp
from jax.experimental import pallas as pl


def kernel(tokens, W_E):
    raise NotImplementedError("write your pallas kernel here")


# speedup vs baseline: 1.4366x; 1.4366x over previous
import jax
import jax.numpy as jnp
from jax import lax
from jax.experimental import pallas as pl
from jax.experimental.pallas import tpu as pltpu
from jax.experimental.pallas import tpu_sc as plsc

D_VOCAB = 100000
D_MODEL = 768
BATCH = 4
SEQ = 4096

NC = 2
NS = 16
NW = NC * NS

TOKENS_TOTAL = BATCH * SEQ
PER_W = TOKENS_TOTAL // NW
CHUNK = 64
CHUNKS = PER_W // CHUNK


def _embed_sc(tokens_w, W_E):
    mesh = plsc.VectorSubcoreMesh(core_axis_name="c", subcore_axis_name="s")

    @pl.kernel(
        mesh=mesh,
        out_type=jax.ShapeDtypeStruct((TOKENS_TOTAL, D_MODEL), jnp.float32),
        scratch_types=[
            pltpu.VMEM((CHUNKS, CHUNK), jnp.int32),
            pltpu.VMEM((CHUNK, D_MODEL), jnp.float32),
            pltpu.SemaphoreType.DMA,
        ],
    )
    def k(tok_hbm, table_hbm, out_hbm, idx_v, rows_v, sem):
        wid = lax.axis_index("s") * NC + lax.axis_index("c")
        base = wid * PER_W
        pltpu.sync_copy(tok_hbm.at[wid], idx_v)
        for j in range(CHUNKS):
            pltpu.async_copy(table_hbm.at[idx_v.at[j]], rows_v, sem).wait()
            pltpu.sync_copy(rows_v, out_hbm.at[pl.ds(base + j * CHUNK, CHUNK)])

    return k(tokens_w, W_E)


def kernel(tokens, W_E):
    tokens_w = tokens.reshape(NW, CHUNKS, CHUNK).astype(jnp.int32)
    emb = _embed_sc(tokens_w, W_E)
    return (tokens, emb.reshape(BATCH, SEQ, D_MODEL))


# speedup vs baseline: 1.5447x; 1.0753x over previous
import jax
import jax.numpy as jnp
from jax import lax
from jax.experimental import pallas as pl
from jax.experimental.pallas import tpu as pltpu
from jax.experimental.pallas import tpu_sc as plsc

D_VOCAB = 100000
D_MODEL = 768
BATCH = 4
SEQ = 4096

NC = 2
NS = 16
NW = NC * NS

TOKENS_TOTAL = BATCH * SEQ
PER_W = TOKENS_TOTAL // NW
CHUNK = 64
CHUNKS = PER_W // CHUNK


def _embed_sc(tokens_w, W_E):
    mesh = plsc.VectorSubcoreMesh(core_axis_name="c", subcore_axis_name="s")

    @pl.kernel(
        mesh=mesh,
        out_type=jax.ShapeDtypeStruct((TOKENS_TOTAL, D_MODEL), jnp.float32),
        scratch_types=[
            pltpu.VMEM((CHUNKS, CHUNK), jnp.int32),
            pltpu.VMEM((CHUNK, D_MODEL), jnp.float32),
            pltpu.VMEM((CHUNK, D_MODEL), jnp.float32),
            pltpu.SemaphoreType.DMA,
            pltpu.SemaphoreType.DMA,
            pltpu.SemaphoreType.DMA,
            pltpu.SemaphoreType.DMA,
        ],
    )
    def k(tok_hbm, table_hbm, out_hbm, idx_v, rows0, rows1, g0, g1, s0, s1):
        wid = lax.axis_index("s") * NC + lax.axis_index("c")
        base = wid * PER_W
        pltpu.sync_copy(tok_hbm.at[wid], idx_v)
        rows, gsem, ssem = [rows0, rows1], [g0, g1], [s0, s1]
        gh, sh = [None, None], [None, None]
        gh[0] = pltpu.async_copy(table_hbm.at[idx_v.at[0]], rows[0], gsem[0])
        for j in range(CHUNKS):
            cur, nxt = j & 1, 1 - (j & 1)
            if j + 1 < CHUNKS:
                if sh[nxt] is not None:
                    sh[nxt].wait()
                gh[nxt] = pltpu.async_copy(
                    table_hbm.at[idx_v.at[j + 1]], rows[nxt], gsem[nxt])
            gh[cur].wait()
            sh[cur] = pltpu.async_copy(
                rows[cur], out_hbm.at[pl.ds(base + j * CHUNK, CHUNK)], ssem[cur])
        sh[0].wait()
        sh[1].wait()

    return k(tokens_w, W_E)


def kernel(tokens, W_E):
    tokens_w = tokens.reshape(NW, CHUNKS, CHUNK).astype(jnp.int32)
    emb = _embed_sc(tokens_w, W_E)
    return (tokens, emb.reshape(BATCH, SEQ, D_MODEL))


# speedup vs baseline: 1.5693x; 1.0159x over previous
import jax
import jax.numpy as jnp
from jax import lax
from jax.experimental import pallas as pl
from jax.experimental.pallas import tpu as pltpu
from jax.experimental.pallas import tpu_sc as plsc

D_VOCAB = 100000
D_MODEL = 768
BATCH = 4
SEQ = 4096

NC = 2
NS = 16
NW = NC * NS

TOKENS_TOTAL = BATCH * SEQ
PER_W = TOKENS_TOTAL // NW
CHUNK = 32
CHUNKS = PER_W // CHUNK
NBUF = 4


def _embed_sc(tokens_w, W_E):
    mesh = plsc.VectorSubcoreMesh(core_axis_name="c", subcore_axis_name="s")

    @pl.kernel(
        mesh=mesh,
        out_type=jax.ShapeDtypeStruct((TOKENS_TOTAL, D_MODEL), jnp.float32),
        scratch_types=(
            [pltpu.VMEM((CHUNKS, CHUNK), jnp.int32)]
            + [pltpu.VMEM((CHUNK, D_MODEL), jnp.float32)] * NBUF
            + [pltpu.SemaphoreType.DMA] * (2 * NBUF)
        ),
    )
    def k(tok_hbm, table_hbm, out_hbm, idx_v, *bufs):
        rows = list(bufs[:NBUF])
        gsem = list(bufs[NBUF:2 * NBUF])
        ssem = list(bufs[2 * NBUF:])
        wid = lax.axis_index("s") * NC + lax.axis_index("c")
        base = wid * PER_W
        pltpu.sync_copy(tok_hbm.at[wid], idx_v)
        gh, sh = [None] * NBUF, [None] * NBUF
        for b in range(NBUF - 1):
            gh[b] = pltpu.async_copy(table_hbm.at[idx_v.at[b]], rows[b], gsem[b])
        for j in range(CHUNKS):
            b = j % NBUF
            jn = j + NBUF - 1
            if jn < CHUNKS:
                bn = jn % NBUF
                if sh[bn] is not None:
                    sh[bn].wait()
                gh[bn] = pltpu.async_copy(
                    table_hbm.at[idx_v.at[jn]], rows[bn], gsem[bn])
            gh[b].wait()
            sh[b] = pltpu.async_copy(
                rows[b], out_hbm.at[pl.ds(base + j * CHUNK, CHUNK)], ssem[b])
        for b in range(NBUF):
            sh[b].wait()

    return k(tokens_w, W_E)


def kernel(tokens, W_E):
    tokens_w = tokens.reshape(NW, CHUNKS, CHUNK).astype(jnp.int32)
    emb = _embed_sc(tokens_w, W_E)
    return (tokens, emb.reshape(BATCH, SEQ, D_MODEL))
